# baseline (speedup 1.0000x reference)
import functools

import jax
import jax.numpy as jnp
from jax import lax
from jax.experimental import pallas as pl
from jax.experimental.pallas import tpu as pltpu
from jax.experimental.pallas import tpu_sc as plsc

_VOCAB = 1000000
_EMBED = 64
_BATCH = 4096
_SEQ = 200

_NC = 2
_NS = 16
_NW = _NC * _NS

_N_WIN = 244


def _make_transpose():
    mesh = plsc.VectorSubcoreMesh(core_axis_name="c", subcore_axis_name="s")

    @functools.partial(
        pl.kernel,
        mesh=mesh,
        out_type=jax.ShapeDtypeStruct((_VOCAB // 2, 128), jnp.float32),
        scratch_types=[
            pltpu.VMEM((2, 64, 128), jnp.float32),
            pltpu.VMEM((2, 64, 128), jnp.float32),
            pltpu.SemaphoreType.DMA((2,)),
            pltpu.SemaphoreType.DMA((2,)),
        ],
        compiler_params=pltpu.CompilerParams(needs_layout_passes=False),
    )
    def transpose_kernel(table_t, lin2, tin, tout, rsem, wsem):
        wid = lax.axis_index("s") * _NC + lax.axis_index("c")
        iota = lax.iota(jnp.int32, 16)
        hrow = iota >> 1
        hcol = (iota & 1) * 64
        row16 = [g * 8 + hrow for g in range(8)]

        def v0_of(i):
            return (wid + 32 * i) * 128

        def read_copy(v0, b):
            return pltpu.make_async_copy(
                table_t.at[:, pl.ds(pl.multiple_of(v0, 128), 128)],
                tin.at[b], rsem.at[b])

        def write_copy(v0, b):
            return pltpu.make_async_copy(
                tout.at[b],
                lin2.at[pl.ds(pl.multiple_of(v0 >> 1, 64), 64)], wsem.at[b])

        def compute(b, gmax=8):
            def e_body(e, carry):
                for g in range(gmax):
                    val = tin[b, e, pl.ds(g * 16, 16)]
                    plsc.store_scatter(tout.at[b], [row16[g], hcol + e], val)
                return carry
            lax.fori_loop(0, 64, e_body, 0)

        def step(i, b, wait_wb, start_read):
            read_copy(v0_of(i), b).wait()
            if start_read:
                read_copy(v0_of(i + 1), 1 - b).start()
            if wait_wb:
                write_copy(v0_of(i - 2), b).wait()
            compute(b)
            write_copy(v0_of(i), b).start()

        read_copy(v0_of(0), 0).start()
        step(0, 0, wait_wb=False, start_read=True)
        step(1, 1, wait_wb=False, start_read=True)

        def pair_body(o, carry):
            i = 2 + 2 * o
            step(i, 0, wait_wb=True, start_read=True)
            step(i + 1, 1, wait_wb=True, start_read=True)
            return carry

        lax.fori_loop(0, (_N_WIN - 4) // 2, pair_body, 0)

        step(_N_WIN - 2, 0, wait_wb=True, start_read=True)
        step(_N_WIN - 1, 1, wait_wb=True, start_read=False)
        write_copy(v0_of(_N_WIN - 2), 0).wait()
        write_copy(v0_of(_N_WIN - 1), 1).wait()

        @pl.when(wid < 4)
        def _extra():
            v0 = (7808 + wid) * 128
            read_copy(v0, 0).start()
            read_copy(v0, 0).wait()
            compute(0)
            write_copy(v0, 0).start()
            write_copy(v0, 0).wait()

        @pl.when(wid == 4)
        def _tail():
            v0 = 7812 * 128
            tail_read = read_copy(v0, 0)
            tail_read.start()
            tail_read.wait()
            compute(0, gmax=4)
            tail_write = pltpu.make_async_copy(
                tout.at[0, pl.ds(0, 32)],
                lin2.at[pl.ds(v0 >> 1, 32)], wsem.at[0])
            tail_write.start()
            tail_write.wait()

    return transpose_kernel


def _make_gather():
    mesh = plsc.VectorSubcoreMesh(core_axis_name="c", subcore_axis_name="s")

    @functools.partial(
        pl.kernel,
        mesh=mesh,
        out_type=jax.ShapeDtypeStruct((_SEQ, _EMBED, _BATCH), jnp.float32),
        scratch_types=[
            pltpu.VMEM((_SEQ, 128), jnp.int32),
            pltpu.VMEM((2, 128), jnp.int32),
            pltpu.VMEM((2, 128, 128), jnp.float32),
            pltpu.VMEM((2, 64, 128), jnp.float32),
            pltpu.SemaphoreType.DMA((2,)),
            pltpu.SemaphoreType.DMA((2,)),
        ],
        compiler_params=pltpu.CompilerParams(needs_layout_passes=False),
    )
    def gather_kernel(lin2, idx_t, out, idxb, jdx, rows_v, stage, gsem, wsem):
        wid = lax.axis_index("s") * _NC + lax.axis_index("c")
        b0 = pl.multiple_of(wid * 128, 128)
        iota = lax.iota(jnp.int32, 16)
        row16 = [g * 16 + iota for g in range(8)]

        pltpu.sync_copy(idx_t.at[:, pl.ds(b0, 128)], idxb)

        def make_jdx(k, b):
            for g in range(8):
                v = idxb[k, pl.ds(g * 16, 16)]
                jdx[b, pl.ds(g * 16, 16)] = v >> 1

        def gather_copy(b):
            return pltpu.make_async_copy(
                lin2.at[jdx.at[b]], rows_v.at[b], gsem.at[b])

        def wb_copy(k, b):
            return pltpu.make_async_copy(
                stage.at[b], out.at[k, :, pl.ds(b0, 128)], wsem.at[b])

        def transpose(k, b):
            pb = []
            for g in range(8):
                v = idxb[k, pl.ds(g * 16, 16)]
                pb.append((v & 1) * 64)

            def e_body(e, carry):
                for g in range(8):
                    val = plsc.load_gather(rows_v.at[b], [row16[g], pb[g] + e])
                    stage[b, e, pl.ds(g * 16, 16)] = val
                return carry
            lax.fori_loop(0, 64, e_body, 0)

        def step(k, b, wait_wb):
            make_jdx(k, b)
            gather_copy(b).start()
            gather_copy(1 - b).wait()
            if wait_wb:
                wb_copy(k - 3, 1 - b).wait()
            transpose(k - 1, 1 - b)
            wb_copy(k - 1, 1 - b).start()

        make_jdx(0, 0)
        gather_copy(0).start()
        step(1, 1, wait_wb=False)
        step(2, 0, wait_wb=False)

        def pair_body(o, carry):
            k = 3 + 2 * o
            step(k, 1, wait_wb=True)
            step(k + 1, 0, wait_wb=True)
            return carry

        lax.fori_loop(0, (_SEQ - 4) // 2, pair_body, 0)

        step(_SEQ - 1, 1, wait_wb=True)
        gather_copy(1).wait()
        wb_copy(_SEQ - 3, 1).wait()
        transpose(_SEQ - 1, 1)
        wb_copy(_SEQ - 1, 1).start()
        wb_copy(_SEQ - 2, 0).wait()
        wb_copy(_SEQ - 1, 1).wait()

    return gather_kernel


@functools.lru_cache(maxsize=None)
def _pipeline():
    return _make_transpose(), _make_gather()


def kernel(indices, table):
    transpose_k, gather_k = _pipeline()
    table_t = table.T
    idx_t = indices.T
    lin2 = transpose_k(table_t)
    out = gather_k(lin2, idx_t)
    return jnp.transpose(out, (2, 0, 1))
